# Initial kernel scaffold; baseline (speedup 1.0000x reference)
import jax
import jax.numpy as jnp
from jax.experimental import pallas as pl


def kernel(disc, cont, history, items, item_table, user_table, city_table, prov_table, dev_table, os_table, W1, b1, W2, b2, W3, b3):
    raise NotImplementedError("write your pallas kernel here")


# speedup vs baseline: 4.7218x; 4.7218x over previous
import functools

import jax
import jax.numpy as jnp
from jax import lax
from jax.experimental import pallas as pl
from jax.experimental.pallas import tpu as pltpu
from jax.experimental.pallas import tpu_sc as plsc

B = 16384
D = 16
H = 50
NI = 200
NIP = 208
NC = 2
NS = 16
NW = NC * NS
RPW = B // NW
HCH = 64
NHC = RPW // HCH
ICH = 16
NIC = RPW // ICH
GPR = NIP // 16

_i32 = jnp.int32
_f32 = jnp.float32


def _lanes():
    return lax.iota(_i32, 16)


def _splat(x):
    return jnp.full((16,), x, _i32)


def _sc_gather_body(discf_h, histf_h, item_h, user_h, prov_h, city_h,
                    user_o, prov_o, city_o, histf_o,
                    discf_v, uidx_v, pidx_v, cidx_v, srows_v, hidx_v, hrows_v,
                    zbuf_v, hbuf_v, t0_v, sem):
    wid = lax.axis_index("s") * NC + lax.axis_index("c")
    base = pl.multiple_of(wid * RPW, RPW)
    lanes = _lanes()

    pltpu.sync_copy(item_h.at[0], t0_v)

    pltpu.sync_copy(discf_h.at[pl.ds(base * 5, RPW * 5)], discf_v)

    def colbody(g, _):
        flat16 = g * 80 + lanes * 5
        for buf, c in ((uidx_v, 0), (pidx_v, 3), (cidx_v, 4)):
            v = plsc.load_gather(discf_v, [flat16 + c])
            buf[pl.ds(pl.multiple_of(g * 16, 16), 16)] = v
        return 0

    lax.fori_loop(0, RPW // 16, colbody, 0)

    for idxbuf, tab, out in ((uidx_v, user_h, user_o), (pidx_v, prov_h, prov_o),
                             (cidx_v, city_h, city_o)):
        cps = [
            pltpu.async_copy(tab.at[idxbuf.at[pl.ds(j * 128, 128)]],
                             srows_v.at[pl.ds(j * 128, 128)], sem)
            for j in range(RPW // 128)
        ]
        for cp in cps:
            cp.wait()
        pltpu.sync_copy(srows_v, out.at[pl.ds(base, RPW)])

    def hist_chunk(cc, _):
        foff = pl.multiple_of(base * H + cc * (HCH * H), 8)
        pltpu.sync_copy(histf_h.at[pl.ds(foff, HCH * H)], hidx_v)
        cps = [
            pltpu.async_copy(item_h.at[hidx_v.at[pl.ds(j * 128, 128)]],
                             hrows_v.at[pl.ds(j * 128, 128)], sem)
            for j in range(HCH * H // 128)
        ]
        for cp in cps:
            cp.wait()

        def zbody(rr, _):
            rbase = pl.multiple_of(rr * 16, 16)
            roff = (rbase + lanes) * H
            zacc = jnp.zeros((16,), _f32)
            for j in range(H):
                iv = plsc.load_gather(hidx_v, [roff + j])
                zacc = zacc + jnp.where(iv == 0, 1.0, 0.0)
            zbuf_v[pl.ds(rbase, 16)] = zacc
            return 0

        lax.fori_loop(0, HCH // 16, zbody, 0)

        def rowbody(r, _):
            off = r * H
            racc = jnp.zeros((16,), _f32)
            for j in range(H):
                racc = racc + hrows_v[off + j, :]
            zr = plsc.load_gather(zbuf_v, [_splat(r)])
            hbuf_v[pl.ds(pl.multiple_of(r * D, D), D)] = \
                (racc - zr * t0_v[:]) * (1.0 / H)
            return 0

        lax.fori_loop(0, HCH, rowbody, 0)
        pltpu.sync_copy(
            hbuf_v, histf_o.at[pl.ds((base + cc * HCH) * D, HCH * D)])
        return 0

    lax.fori_loop(0, NHC, hist_chunk, 0)


@functools.cache
def _sc_gather():
  return pl.kernel(
    _sc_gather_body,
    out_type=[jax.ShapeDtypeStruct((B, D), _f32)] * 3
    + [jax.ShapeDtypeStruct((B * D,), _f32)],
    mesh=plsc.VectorSubcoreMesh(core_axis_name="c", subcore_axis_name="s",
                                num_cores=NC, num_subcores=NS),
    compiler_params=pltpu.CompilerParams(needs_layout_passes=False, use_tc_tiling_on_sc=False),
    scratch_types=[
        pltpu.VMEM((RPW * 5,), _i32),
        pltpu.VMEM((RPW,), _i32),
        pltpu.VMEM((RPW,), _i32),
        pltpu.VMEM((RPW,), _i32),
        pltpu.VMEM((RPW, D), _f32),
        pltpu.VMEM((HCH * H,), _i32),
        pltpu.VMEM((HCH * H, D), _f32),
        pltpu.VMEM((HCH,), _f32),
        pltpu.VMEM((HCH * D,), _f32),
        pltpu.VMEM((16,), _f32),
        pltpu.SemaphoreType.DMA,
    ],
  )


def _sc_items_body(itemsf_h, item_h, uf_h,
                   dotf_o, nb2f_o,
                   u_v, iidx_v, irows_v, tbuf_v, dot_v, nb2_v, sem):
    wid = lax.axis_index("s") * NC + lax.axis_index("c")
    base = pl.multiple_of(wid * RPW, RPW)
    lanes = _lanes()
    npad = ICH * NI

    pltpu.sync_copy(uf_h.at[pl.ds(base * D, RPW * D)], u_v)

    iidx_v[pl.ds(npad, 16)] = jnp.zeros((16,), _i32)
    for j in range(16):
        irows_v[npad + j, :] = jnp.zeros((16,), _f32)

    def chunk(cc, _):
        foff = pl.multiple_of(base * NI + cc * npad, 8)
        pltpu.sync_copy(itemsf_h.at[pl.ds(foff, npad)],
                        iidx_v.at[pl.ds(0, npad)])
        cps = [
            pltpu.async_copy(item_h.at[iidx_v.at[pl.ds(j * 128, 128)]],
                             irows_v.at[pl.ds(j * 128, 128)], sem)
            for j in range(npad // 128)
        ]
        for cp in cps:
            cp.wait()

        def rowbody(r, _):
            rbase = cc * ICH + r
            ub = [plsc.load_gather(u_v, [_splat(rbase * D + d)])
                  for d in range(D)]
            for g in range(GPR):
                goff = pl.multiple_of(r * NI, 8) + g * 16
                idxv = iidx_v[pl.ds(goff, 16)]
                for k in range(16):
                    rv = irows_v[goff + k, :]
                    plsc.store_scatter(tbuf_v, [lanes * 16 + k], rv)
                dotacc = jnp.zeros((16,), _f32)
                nbacc = jnp.zeros((16,), _f32)
                for d in range(D):
                    colv = tbuf_v[pl.ds(d * 16, 16)]
                    dotacc = dotacc + colv * ub[d]
                    nbacc = nbacc + colv * colv
                m = idxv != 0
                doff = pl.multiple_of(r * NIP, 16) + g * 16
                dot_v[pl.ds(doff, 16)] = jnp.where(m, dotacc, 0.0)
                nb2_v[pl.ds(doff, 16)] = jnp.where(m, nbacc, 0.0)
            return 0

        lax.fori_loop(0, ICH, rowbody, 0)
        pltpu.sync_copy(dot_v,
                        dotf_o.at[pl.ds((base + cc * ICH) * NIP, ICH * NIP)])
        pltpu.sync_copy(nb2_v,
                        nb2f_o.at[pl.ds((base + cc * ICH) * NIP, ICH * NIP)])
        return 0

    lax.fori_loop(0, NIC, chunk, 0)


@functools.cache
def _sc_items():
  return pl.kernel(
    _sc_items_body,
    out_type=[jax.ShapeDtypeStruct((B * NIP,), _f32)] * 2,
    mesh=plsc.VectorSubcoreMesh(core_axis_name="c", subcore_axis_name="s",
                                num_cores=NC, num_subcores=NS),
    compiler_params=pltpu.CompilerParams(needs_layout_passes=False, use_tc_tiling_on_sc=False),
    scratch_types=[
        pltpu.VMEM((RPW * D,), _f32),
        pltpu.VMEM((ICH * NI + 16,), _i32),
        pltpu.VMEM((ICH * NI + 16, D), _f32),
        pltpu.VMEM((256,), _f32),
        pltpu.VMEM((ICH * NIP,), _f32),
        pltpu.VMEM((ICH * NIP,), _f32),
        pltpu.SemaphoreType.DMA,
    ],
  )


_MBLK = 2048


def _mlp_body(user_r, prov_r, city_r, hist_r, cont_r,
              w1_r, b1_r, w2_r, b2_r, w3_r, b3_r, out_r):
    w1 = w1_r[...]
    h = (jnp.dot(user_r[...], w1[0:16, :], preferred_element_type=_f32)
         + jnp.dot(prov_r[...], w1[16:32, :], preferred_element_type=_f32)
         + jnp.dot(city_r[...], w1[32:48, :], preferred_element_type=_f32)
         + jnp.dot(hist_r[...], w1[48:64, :], preferred_element_type=_f32)
         + jnp.dot(cont_r[...], w1[64:92, :], preferred_element_type=_f32)
         + b1_r[...])
    h = jnp.dot(h, w2_r[...], preferred_element_type=_f32) + b2_r[...]
    out_r[...] = jnp.dot(h, w3_r[...], preferred_element_type=_f32) + b3_r[...]


def _row_spec(cols):
    return pl.BlockSpec((_MBLK, cols), lambda i: (i, 0))


def _full_spec(shape):
    return pl.BlockSpec(shape, lambda i: tuple(0 for _ in shape))


_tc_mlp = pl.pallas_call(
    _mlp_body,
    grid=(B // _MBLK,),
    in_specs=[
        _row_spec(D), _row_spec(D), _row_spec(D), _row_spec(D), _row_spec(28),
        _full_spec((92, 128)), _full_spec((1, 128)),
        _full_spec((128, 32)), _full_spec((1, 32)),
        _full_spec((32, 16)), _full_spec((1, 16)),
    ],
    out_specs=_row_spec(D),
    out_shape=jax.ShapeDtypeStruct((B, D), _f32),
)


def _softmax_body(dot_r, nb2_r, u_r, out_r):
    dots = dot_r[:, :NI]
    nb2 = nb2_r[:, :NI]
    u = u_r[...]
    na2 = jnp.sum(u * u, axis=1, keepdims=True)
    den = jnp.maximum(jnp.sqrt(na2 * nb2), 1e-8)
    logits = dots / den
    m = jnp.max(logits, axis=1, keepdims=True)
    e = jnp.exp(logits - m)
    out_r[...] = e / jnp.sum(e, axis=1, keepdims=True)


_tc_softmax = pl.pallas_call(
    _softmax_body,
    grid=(B // _MBLK,),
    in_specs=[_row_spec(NIP), _row_spec(NIP), _row_spec(D)],
    out_specs=_row_spec(NI),
    out_shape=jax.ShapeDtypeStruct((B, NI), _f32),
)


def kernel(disc, cont, history, items, item_table, user_table, city_table,
           prov_table, dev_table, os_table, W1, b1, W2, b2, W3, b3):
    user_e, prov_e, city_e, histf_e = _sc_gather()(
        disc.reshape(-1), history.reshape(-1), item_table, user_table,
        prov_table, city_table)
    u16 = _tc_mlp(user_e, prov_e, city_e, histf_e.reshape(B, D), cont,
                  W1, b1.reshape(1, -1), W2, b2.reshape(1, -1),
                  W3, b3.reshape(1, -1))
    dotf_p, nb2f_p = _sc_items()(items.reshape(-1), item_table,
                                 u16.reshape(-1))
    return _tc_softmax(dotf_p.reshape(B, NIP), nb2f_p.reshape(B, NIP), u16)
